# baseline (speedup 1.0000x reference)
import functools

import jax
import jax.numpy as jnp
from jax import lax
from jax.experimental import pallas as pl
from jax.experimental.pallas import tpu as pltpu
from jax.experimental.pallas import tpu_sc as plsc

_LANES = 16
_IDX_CHUNK = 128


def _sc_sampled(wt, bias, sids, V, D):
    S = sids.shape[0]
    info = plsc.get_sparse_core_info()
    nc, ns = info.num_cores, info.num_subcores
    nw = nc * ns
    spw = S // nw
    BLK = 128

    mesh = plsc.VectorSubcoreMesh(core_axis_name="c", subcore_axis_name="s")

    @functools.partial(
        pl.kernel,
        mesh=mesh,
        out_type=(
            jax.ShapeDtypeStruct((S * D,), jnp.float32),
            jax.ShapeDtypeStruct((S,), jnp.float32),
        ),
        compiler_params=pltpu.CompilerParams(needs_layout_passes=False),
        scratch_types=[
            pltpu.VMEM((spw,), jnp.int32),
            pltpu.VMEM((D, BLK), jnp.float32),
            pltpu.VMEM((D, BLK), jnp.float32),
            pltpu.VMEM((spw * D,), jnp.float32),
            pltpu.VMEM((spw,), jnp.float32),
            pltpu.SemaphoreType.DMA,
            pltpu.SemaphoreType.DMA,
            pltpu.SemaphoreType.DMA,
        ],
    )
    def k(wt_hbm, b_hbm, sid_hbm, swf_hbm, sb_hbm,
          sidx, blk0, blk1, rows, sbv, sem0, sem1, bsem):
        wid = lax.axis_index("s") * nc + lax.axis_index("c")
        base = wid * spw
        pltpu.sync_copy(sid_hbm.at[pl.ds(base, spw)], sidx)
        bc = pltpu.async_copy(b_hbm.at[sidx], sbv, bsem)

        blks = (blk0, blk1)
        sems = (sem0, sem1)
        ids_vec = [sidx[pl.ds(g * _LANES, _LANES)]
                   for g in range(spw // _LANES)]
        ids = [v[j] for v in ids_vec for j in range(_LANES)]

        def blk_off(i):
            return pl.multiple_of((ids[i] // BLK) * BLK, BLK)

        pltpu.async_copy(
            wt_hbm.at[:, pl.ds(blk_off(0), BLK)], blks[0], sems[0])
        lane_iota = lax.iota(jnp.int32, _LANES)
        for i in range(spw):
            if i + 1 < spw:
                pltpu.async_copy(
                    wt_hbm.at[:, pl.ds(blk_off(i + 1), BLK)],
                    blks[(i + 1) % 2], sems[(i + 1) % 2])
            pltpu.make_async_copy(
                wt_hbm.at[:, pl.ds(0, BLK)], blks[i % 2], sems[i % 2]).wait()
            col = jnp.broadcast_to(ids[i] % BLK, (_LANES,))
            for g in range(D // _LANES):
                rowi = g * _LANES + lane_iota
                vals = plsc.load_gather(blks[i % 2], [rowi, col])
                rows[pl.ds(i * D + g * _LANES, _LANES)] = vals

        pltpu.sync_copy(rows, swf_hbm.at[pl.ds(base * D, spw * D)])
        bc.wait()
        pltpu.sync_copy(sbv, sb_hbm.at[pl.ds(base, spw)])

    return k(wt, bias, sids)


def _sc_true(W, bias, tids):
    B = tids.shape[0]
    D = W.shape[1]
    info = plsc.get_sparse_core_info()
    nc, ns = info.num_cores, info.num_subcores
    nw = nc * ns
    bt = B // nw

    mesh = plsc.VectorSubcoreMesh(core_axis_name="c", subcore_axis_name="s")

    @functools.partial(
        pl.kernel,
        mesh=mesh,
        out_type=(
            jax.ShapeDtypeStruct((B, D), jnp.float32),
            jax.ShapeDtypeStruct((B,), jnp.float32),
        ),
        compiler_params=pltpu.CompilerParams(use_tc_tiling_on_sc=False),
        scratch_types=[
            pltpu.VMEM((bt,), jnp.int32),
            pltpu.VMEM((bt, D), jnp.float32),
            pltpu.VMEM((bt,), jnp.float32),
            pltpu.SemaphoreType.DMA,
            pltpu.SemaphoreType.DMA,
        ],
    )
    def k(w_hbm, b_hbm, tid_hbm, tw_hbm, tb_hbm,
          tidx, trows, tbv, sem, bsem):
        wid = lax.axis_index("s") * nc + lax.axis_index("c")
        base = wid * bt
        pltpu.sync_copy(tid_hbm.at[pl.ds(base, bt)], tidx)
        copies = []
        for j in range(bt // _IDX_CHUNK):
            sl = pl.ds(j * _IDX_CHUNK, _IDX_CHUNK)
            copies.append(pltpu.async_copy(
                w_hbm.at[tidx.at[sl]], trows.at[sl], sem))
            copies.append(pltpu.async_copy(
                b_hbm.at[tidx.at[sl]], tbv.at[sl], bsem))
        for c in copies:
            c.wait()
        pltpu.sync_copy(trows, tw_hbm.at[pl.ds(base, bt)])
        pltpu.sync_copy(tbv, tb_hbm.at[pl.ds(base, bt)])

    return k(W, bias, tids)


def _tc_main(xt, rhs, N, bm=1024):
    D, B = xt.shape
    K = rhs.shape[0]

    def body(xt_ref, rhs_ref, out_ref):
        xtb = xt_ref[...]
        ones = jnp.ones((1, bm), jnp.float32)
        lhs = jnp.concatenate([xtb, ones], axis=0)
        out_ref[...] = lax.dot_general(
            lhs, rhs_ref[...], (((0,), (0,)), ((), ())),
            preferred_element_type=jnp.float32)

    return pl.pallas_call(
        body,
        grid=(B // bm,),
        in_specs=[
            pl.BlockSpec((D, bm), lambda i: (0, i)),
            pl.BlockSpec((K, N), lambda i: (0, 0)),
        ],
        out_specs=pl.BlockSpec((bm, N), lambda i: (i, 0)),
        out_shape=jax.ShapeDtypeStruct((B, N), jnp.float32),
    )(xt, rhs)


def _tc_fix0(out_prev, x, tw, tb, bm=1024):
    B, N = out_prev.shape
    D = x.shape[1]

    def body(x_ref, tw_ref, tb_ref, blk_ref, out_ref):
        tl = jnp.sum(x_ref[...] * tw_ref[...], axis=1, keepdims=True)
        tl = tl + tb_ref[...].reshape(bm, 1)
        lane = lax.broadcasted_iota(jnp.int32, (bm, 128), 1)
        out_ref[...] = jnp.where(lane == 0, tl, blk_ref[...])

    return pl.pallas_call(
        body,
        grid=(B // bm,),
        in_specs=[
            pl.BlockSpec((bm, D), lambda i: (i, 0)),
            pl.BlockSpec((bm, D), lambda i: (i, 0)),
            pl.BlockSpec((bm,), lambda i: (i,)),
            pl.BlockSpec((bm, 128), lambda i: (i, 0)),
        ],
        out_specs=pl.BlockSpec((bm, 128), lambda i: (i, 0)),
        out_shape=jax.ShapeDtypeStruct((B, N), jnp.float32),
        input_output_aliases={3: 0},
    )(x, tw, tb, out_prev)


def kernel(inputs, W, bias, target, sampled):
    V, D = W.shape
    tids = target.reshape(-1)
    S = sampled.shape[0]
    wt = W.T
    xt = inputs.T

    swflat, sb = _sc_sampled(wt, bias, sampled, V, D)
    sw = swflat.reshape(S, D)
    rhs = jnp.concatenate([
        jnp.concatenate([jnp.zeros((D, 1), jnp.float32), sw.T], axis=1),
        jnp.concatenate(
            [jnp.zeros((1, 1), jnp.float32), sb[None, :]], axis=1),
    ], axis=0)

    tw, tb = _sc_true(W, bias, tids)
    out1 = _tc_main(xt, rhs, S + 1)
    return _tc_fix0(out1, inputs, tw, tb)
